# baseline (speedup 1.0000x reference)
import functools

import jax
import jax.numpy as jnp
from jax.experimental import pallas as pl
from jax.experimental.pallas import tpu as pltpu

_NUM_CLASSES = 81
_THRESHOLD = 0.5
_NEGPOS_RATIO = 3
_V0 = 0.1
_V1 = 0.2
_R = 100
_L = 200


def _smooth_l1(d):
    ad = jnp.abs(d)
    return jnp.where(ad < 1.0, 0.5 * ad * ad, ad - 0.5)


def _mbl_body(targets_ref, loc_ref, conf_ref, priors_ref,
              ll_ref, lc_ref, np_ref, *, num_t, num_p):
    f32 = jnp.float32
    i32 = jnp.int32

    pcx = priors_ref[0]
    pcy = priors_ref[1]
    pw = priors_ref[2]
    ph = priors_ref[3]
    px1 = pcx - pw * 0.5
    py1 = pcy - ph * 0.5
    px2 = pcx + pw * 0.5
    py2 = pcy + ph * 0.5
    parea = (px2 - px1) * (py2 - py1)

    pidx = (jax.lax.broadcasted_iota(i32, (_R, _L), 0) * _L
            + jax.lax.broadcasted_iota(i32, (_R, _L), 1))

    bto = jnp.full((_R, _L), -1.0, dtype=f32)
    bti = jnp.zeros((_R, _L), dtype=i32)
    best_prior = []
    tcoords = []
    for t in range(num_t):
        tx1 = targets_ref[0, t, 0]
        ty1 = targets_ref[0, t, 1]
        tx2 = targets_ref[0, t, 2]
        ty2 = targets_ref[0, t, 3]
        tlb = targets_ref[0, t, 4]
        tcoords.append((tx1, ty1, tx2, ty2, tlb))
        iw = jnp.maximum(jnp.minimum(tx2, px2) - jnp.maximum(tx1, px1), 0.0)
        ih = jnp.maximum(jnp.minimum(ty2, py2) - jnp.maximum(ty1, py1), 0.0)
        inter = iw * ih
        tarea = (tx2 - tx1) * (ty2 - ty1)
        ov = inter / (tarea + parea - inter)
        upd = ov > bto
        bti = jnp.where(upd, t, bti)
        bto = jnp.where(upd, ov, bto)
        m = jnp.max(ov)
        bp = jnp.min(jnp.where(ov == m, pidx, num_p))
        best_prior.append(bp)

    for t in range(num_t):
        hit = pidx == best_prior[t]
        bto = jnp.where(hit, 2.0, bto)
        bti = jnp.where(hit, t, bti)

    mx1 = jnp.zeros((_R, _L), dtype=f32)
    my1 = jnp.zeros((_R, _L), dtype=f32)
    mx2 = jnp.zeros((_R, _L), dtype=f32)
    my2 = jnp.zeros((_R, _L), dtype=f32)
    mlb = jnp.zeros((_R, _L), dtype=f32)
    for t in range(num_t):
        sel = bti == t
        tx1, ty1, tx2, ty2, tlb = tcoords[t]
        mx1 = jnp.where(sel, tx1, mx1)
        my1 = jnp.where(sel, ty1, my1)
        mx2 = jnp.where(sel, tx2, mx2)
        my2 = jnp.where(sel, ty2, my2)
        mlb = jnp.where(sel, tlb, mlb)

    pos = bto >= _THRESHOLD
    conf_i = jnp.where(pos, (mlb + 1.0).astype(i32), 0)

    g_cx = ((mx1 + mx2) * 0.5 - pcx) / (_V0 * pw)
    g_cy = ((my1 + my2) * 0.5 - pcy) / (_V0 * ph)
    g_w = jnp.log((mx2 - mx1) / pw) / _V1
    g_h = jnp.log((my2 - my1) / ph) / _V1
    sl = (_smooth_l1(loc_ref[0, 0] - g_cx)
          + _smooth_l1(loc_ref[0, 1] - g_cy)
          + _smooth_l1(loc_ref[0, 2] - g_w)
          + _smooth_l1(loc_ref[0, 3] - g_h))
    loss_l = jnp.sum(jnp.where(pos, sl, 0.0))

    x = conf_ref[0]
    xm = jnp.max(x, axis=-1)
    lse = jnp.log(jnp.sum(jnp.exp(x - xm[..., None]), axis=-1)) + xm
    cio = jax.lax.broadcasted_iota(i32, (_R, _L, _NUM_CLASSES), 2)
    gath = jnp.sum(jnp.where(cio == conf_i[..., None], x, 0.0), axis=-1)
    ce = lse - gath

    num_pos = jnp.sum(pos.astype(i32))
    sum_pos_ce = jnp.sum(jnp.where(pos, ce, 0.0))

    s = jnp.where(pos, 0.0, ce)
    bits = jax.lax.bitcast_convert_type(s, i32)
    k = jnp.minimum(_NEGPOS_RATIO * num_pos, num_p - 1)

    def bs_body(_, lohi):
        lo, hi = lohi
        mid = lo + (hi - lo + 1) // 2
        cnt = jnp.sum((bits >= mid).astype(i32))
        ok = cnt >= k
        return jnp.where(ok, mid, lo), jnp.where(ok, hi, mid - 1)

    lo, _ = jax.lax.fori_loop(0, 31, bs_body,
                              (jnp.int32(0), jnp.int32(0x7F7FFFFF)))
    tau = jax.lax.bitcast_convert_type(lo, f32)
    gt = bits > lo
    cnt_gt = jnp.sum(gt.astype(i32))
    sum_top = (jnp.sum(jnp.where(gt, s, 0.0))
               + tau * (k - cnt_gt).astype(f32))
    sum_top = jnp.where(k > 0, sum_top, 0.0)

    ll_ref[0, 0, 0] = loss_l
    lc_ref[0, 0, 0] = sum_pos_ce + sum_top
    np_ref[0, 0, 0] = num_pos.astype(f32)


def kernel(loc_data, conf_data, priors, targets):
    B, P, C = conf_data.shape
    T = targets.shape[1]
    loc_r = loc_data.transpose(0, 2, 1).reshape(B, 4, _R, _L)
    conf_r = conf_data.reshape(B, _R, _L, C)
    priors_r = priors.T.reshape(4, _R, _L)

    body = functools.partial(_mbl_body, num_t=T, num_p=P)
    out_shape = [jax.ShapeDtypeStruct((B, 1, 1), jnp.float32)] * 3
    ll, lc, npos = pl.pallas_call(
        body,
        grid=(B,),
        in_specs=[
            pl.BlockSpec((1, T, 5), lambda b: (b, 0, 0),
                         memory_space=pltpu.SMEM),
            pl.BlockSpec((1, 4, _R, _L), lambda b: (b, 0, 0, 0)),
            pl.BlockSpec((1, _R, _L, C), lambda b: (b, 0, 0, 0)),
            pl.BlockSpec((4, _R, _L), lambda b: (0, 0, 0)),
        ],
        out_specs=[
            pl.BlockSpec((1, 1, 1), lambda b: (b, 0, 0),
                         memory_space=pltpu.SMEM),
            pl.BlockSpec((1, 1, 1), lambda b: (b, 0, 0),
                         memory_space=pltpu.SMEM),
            pl.BlockSpec((1, 1, 1), lambda b: (b, 0, 0),
                         memory_space=pltpu.SMEM),
        ],
        out_shape=out_shape,
        compiler_params=pltpu.CompilerParams(
            dimension_semantics=("arbitrary",)),
    )(targets, loc_r, conf_r, priors_r)

    N = jnp.maximum(jnp.sum(npos), 1.0)
    return jnp.sum(ll) / N, jnp.sum(lc) / N


# speedup vs baseline: 1.1280x; 1.1280x over previous
import functools

import jax
import jax.numpy as jnp
from jax.experimental import pallas as pl
from jax.experimental.pallas import tpu as pltpu

_NUM_CLASSES = 81
_THRESHOLD = 0.5
_NEGPOS_RATIO = 3
_V0 = 0.1
_V1 = 0.2
_R = 160
_L = 125


def _smooth_l1(d):
    ad = jnp.abs(d)
    return jnp.where(ad < 1.0, 0.5 * ad * ad, ad - 0.5)


def _mbl_body(targets_ref, loc_ref, conf_ref, priors_ref,
              ll_ref, lc_ref, np_ref, *, num_t, num_p):
    f32 = jnp.float32
    i32 = jnp.int32

    pcx = priors_ref[0]
    pcy = priors_ref[1]
    pw = priors_ref[2]
    ph = priors_ref[3]
    px1 = pcx - pw * 0.5
    py1 = pcy - ph * 0.5
    px2 = pcx + pw * 0.5
    py2 = pcy + ph * 0.5
    parea = (px2 - px1) * (py2 - py1)

    pidx = (jax.lax.broadcasted_iota(i32, (_R, _L), 0) * _L
            + jax.lax.broadcasted_iota(i32, (_R, _L), 1))

    bto = jnp.full((_R, _L), -1.0, dtype=f32)
    bti = jnp.zeros((_R, _L), dtype=i32)
    best_prior = []
    tcoords = []
    for t in range(num_t):
        tx1 = targets_ref[0, t, 0]
        ty1 = targets_ref[0, t, 1]
        tx2 = targets_ref[0, t, 2]
        ty2 = targets_ref[0, t, 3]
        tlb = targets_ref[0, t, 4]
        tcoords.append((tx1, ty1, tx2, ty2, tlb))
        iw = jnp.maximum(jnp.minimum(tx2, px2) - jnp.maximum(tx1, px1), 0.0)
        ih = jnp.maximum(jnp.minimum(ty2, py2) - jnp.maximum(ty1, py1), 0.0)
        inter = iw * ih
        tarea = (tx2 - tx1) * (ty2 - ty1)
        ov = inter / (tarea + parea - inter)
        upd = ov > bto
        bti = jnp.where(upd, t, bti)
        bto = jnp.where(upd, ov, bto)
        m = jnp.max(ov)
        bp = jnp.min(jnp.where(ov == m, pidx, num_p))
        best_prior.append(bp)

    for t in range(num_t):
        hit = pidx == best_prior[t]
        bto = jnp.where(hit, 2.0, bto)
        bti = jnp.where(hit, t, bti)

    mx1 = jnp.zeros((_R, _L), dtype=f32)
    my1 = jnp.zeros((_R, _L), dtype=f32)
    mx2 = jnp.zeros((_R, _L), dtype=f32)
    my2 = jnp.zeros((_R, _L), dtype=f32)
    mlb = jnp.zeros((_R, _L), dtype=f32)
    for t in range(num_t):
        sel = bti == t
        tx1, ty1, tx2, ty2, tlb = tcoords[t]
        mx1 = jnp.where(sel, tx1, mx1)
        my1 = jnp.where(sel, ty1, my1)
        mx2 = jnp.where(sel, tx2, mx2)
        my2 = jnp.where(sel, ty2, my2)
        mlb = jnp.where(sel, tlb, mlb)

    pos = bto >= _THRESHOLD
    conf_i = jnp.where(pos, (mlb + 1.0).astype(i32), 0)

    g_cx = ((mx1 + mx2) * 0.5 - pcx) / (_V0 * pw)
    g_cy = ((my1 + my2) * 0.5 - pcy) / (_V0 * ph)
    g_w = jnp.log((mx2 - mx1) / pw) / _V1
    g_h = jnp.log((my2 - my1) / ph) / _V1
    sl = (_smooth_l1(loc_ref[0, 0] - g_cx)
          + _smooth_l1(loc_ref[0, 1] - g_cy)
          + _smooth_l1(loc_ref[0, 2] - g_w)
          + _smooth_l1(loc_ref[0, 3] - g_h))
    loss_l = jnp.sum(jnp.where(pos, sl, 0.0))

    x = conf_ref[0]
    xm = jnp.max(x, axis=-1)
    lse = jnp.log(jnp.sum(jnp.exp(x - xm[..., None]), axis=-1)) + xm
    cio = jax.lax.broadcasted_iota(i32, (_R, _L, _NUM_CLASSES), 2)
    gath = jnp.sum(jnp.where(cio == conf_i[..., None], x, 0.0), axis=-1)
    ce = lse - gath

    num_pos = jnp.sum(pos.astype(i32))
    sum_pos_ce = jnp.sum(jnp.where(pos, ce, 0.0))

    s = jnp.where(pos, 0.0, ce)
    bits = jax.lax.bitcast_convert_type(s, i32)
    k = jnp.minimum(_NEGPOS_RATIO * num_pos, num_p - 1)

    def bs_body(_, lohi):
        lo, hi = lohi
        mid = lo + (hi - lo + 1) // 2
        cnt = jnp.sum((bits >= mid).astype(i32))
        ok = cnt >= k
        return jnp.where(ok, mid, lo), jnp.where(ok, hi, mid - 1)

    lo, _ = jax.lax.fori_loop(0, 31, bs_body,
                              (jnp.int32(0), jnp.int32(0x7F7FFFFF)))
    tau = jax.lax.bitcast_convert_type(lo, f32)
    gt = bits > lo
    cnt_gt = jnp.sum(gt.astype(i32))
    sum_top = (jnp.sum(jnp.where(gt, s, 0.0))
               + tau * (k - cnt_gt).astype(f32))
    sum_top = jnp.where(k > 0, sum_top, 0.0)

    ll_ref[0, 0, 0] = loss_l
    lc_ref[0, 0, 0] = sum_pos_ce + sum_top
    np_ref[0, 0, 0] = num_pos.astype(f32)


def kernel(loc_data, conf_data, priors, targets):
    B, P, C = conf_data.shape
    T = targets.shape[1]
    loc_r = loc_data.transpose(0, 2, 1).reshape(B, 4, _R, _L)
    conf_r = conf_data.reshape(B, _R, _L, C)
    priors_r = priors.T.reshape(4, _R, _L)

    body = functools.partial(_mbl_body, num_t=T, num_p=P)
    out_shape = [jax.ShapeDtypeStruct((B, 1, 1), jnp.float32)] * 3
    ll, lc, npos = pl.pallas_call(
        body,
        grid=(B,),
        in_specs=[
            pl.BlockSpec((1, T, 5), lambda b: (b, 0, 0),
                         memory_space=pltpu.SMEM),
            pl.BlockSpec((1, 4, _R, _L), lambda b: (b, 0, 0, 0)),
            pl.BlockSpec((1, _R, _L, C), lambda b: (b, 0, 0, 0)),
            pl.BlockSpec((4, _R, _L), lambda b: (0, 0, 0)),
        ],
        out_specs=[
            pl.BlockSpec((1, 1, 1), lambda b: (b, 0, 0),
                         memory_space=pltpu.SMEM),
            pl.BlockSpec((1, 1, 1), lambda b: (b, 0, 0),
                         memory_space=pltpu.SMEM),
            pl.BlockSpec((1, 1, 1), lambda b: (b, 0, 0),
                         memory_space=pltpu.SMEM),
        ],
        out_shape=out_shape,
        compiler_params=pltpu.CompilerParams(
            dimension_semantics=("arbitrary",)),
    )(targets, loc_r, conf_r, priors_r)

    N = jnp.maximum(jnp.sum(npos), 1.0)
    return jnp.sum(ll) / N, jnp.sum(lc) / N
